# baseline (speedup 1.0000x reference)
import functools

import jax
import jax.numpy as jnp
from jax import lax
from jax.experimental import pallas as pl
from jax.experimental.pallas import tpu as pltpu
from jax.experimental.pallas import tpu_sc as plsc

_V = 1000
_D = 128
_CHUNK = 64


def _gram_body(t_ref, g_ref):
    t = t_ref[...]
    g_ref[...] = lax.dot_general(
        t, t, (((1,), (1,)), ((), ())), preferred_element_type=jnp.float32
    )


def _gram(table):
    return pl.pallas_call(
        _gram_body,
        out_shape=jax.ShapeDtypeStruct((_V, _V), jnp.float32),
    )(table)


@functools.cache
def _gather_fn(n_tok):
    info = plsc.get_sparse_core_info()
    nc, ns = info.num_cores, info.num_subcores
    nw = nc * ns
    per_w = n_tok // nw
    n_steps = per_w // _CHUNK
    assert per_w * nw == n_tok and n_steps * _CHUNK == per_w
    mesh = plsc.VectorSubcoreMesh(core_axis_name="c", subcore_axis_name="s")

    @functools.partial(
        pl.kernel,
        mesh=mesh,
        compiler_params=pltpu.CompilerParams(use_tc_tiling_on_sc=False),
        out_type=jax.ShapeDtypeStruct((n_tok, _V), jnp.float32),
        scratch_types=[
            pltpu.VMEM((per_w,), jnp.int32),
            pltpu.VMEM((_CHUNK, _V), jnp.float32),
            pltpu.SemaphoreType.DMA,
        ],
    )
    def gather(gram_hbm, ids_hbm, out_hbm, idx_v, buf, sem):
        wid = lax.axis_index("s") * nc + lax.axis_index("c")
        base = wid * per_w
        pltpu.sync_copy(ids_hbm.at[pl.ds(base, per_w)], idx_v)

        def step(i, carry):
            pltpu.async_copy(
                gram_hbm.at[idx_v.at[pl.ds(i * _CHUNK, _CHUNK)]], buf, sem
            ).wait()
            pltpu.sync_copy(buf, out_hbm.at[pl.ds(base + i * _CHUNK, _CHUNK)])
            return carry

        lax.fori_loop(0, n_steps, step, 0)

    return gather


def kernel(ids, table):
    b, l = ids.shape
    gram = _gram(table)
    flat = _gather_fn(b * l)(gram, ids.reshape(-1))
    return flat.reshape(b, l, _V)


# speedup vs baseline: 1.0205x; 1.0205x over previous
import functools

import jax
import jax.numpy as jnp
from jax import lax
from jax.experimental import pallas as pl
from jax.experimental.pallas import tpu as pltpu
from jax.experimental.pallas import tpu_sc as plsc

_V = 1000
_D = 128


def _gram_body(t_ref, g_ref):
    t = t_ref[...]
    g_ref[...] = lax.dot_general(
        t, t, (((1,), (1,)), ((), ())), preferred_element_type=jnp.float32
    )


def _gram(table):
    return pl.pallas_call(
        _gram_body,
        out_shape=jax.ShapeDtypeStruct((_V, _V), jnp.float32),
    )(table)


@functools.cache
def _gather_fn(b, l):
    info = plsc.get_sparse_core_info()
    nc, ns = info.num_cores, info.num_subcores
    nw = nc * ns
    nb = b // nw
    assert nb * nw == b and nb % 2 == 0
    mesh = plsc.VectorSubcoreMesh(core_axis_name="c", subcore_axis_name="s")

    @functools.partial(
        pl.kernel,
        mesh=mesh,
        compiler_params=pltpu.CompilerParams(use_tc_tiling_on_sc=False),
        out_type=jax.ShapeDtypeStruct((b, l, _V), jnp.float32),
        scratch_types=[
            pltpu.VMEM((nb, l), jnp.int32),
            pltpu.VMEM((l, _V), jnp.float32),
            pltpu.VMEM((l, _V), jnp.float32),
            pltpu.SemaphoreType.DMA,
            pltpu.SemaphoreType.DMA,
            pltpu.SemaphoreType.DMA,
            pltpu.SemaphoreType.DMA,
        ],
    )
    def gather(gram_hbm, ids_hbm, out_hbm, idx_v, buf_a, buf_b, ga, gb, wa, wb):
        wid = lax.axis_index("s") * nc + lax.axis_index("c")
        base = wid * nb
        pltpu.sync_copy(ids_hbm.at[pl.ds(base, nb)], idx_v)

        def g_issue(i, buf, sem):
            pltpu.async_copy(gram_hbm.at[idx_v.at[i]], buf, sem)

        def g_wait(i, buf, sem):
            pltpu.make_async_copy(gram_hbm.at[idx_v.at[i]], buf, sem).wait()

        def w_issue(i, buf, sem):
            pltpu.async_copy(buf, out_hbm.at[base + i], sem)

        def w_wait(i, buf, sem):
            pltpu.make_async_copy(buf, out_hbm.at[base + i], sem).wait()

        g_issue(0, buf_a, ga)
        g_issue(1, buf_b, gb)

        def body(j, carry):
            ia = 2 * j
            ib = ia + 1
            g_wait(ia, buf_a, ga)
            w_issue(ia, buf_a, wa)
            g_wait(ib, buf_b, gb)
            w_wait(ia, buf_a, wa)

            @pl.when(ia + 2 < nb)
            def _():
                g_issue(ia + 2, buf_a, ga)

            w_issue(ib, buf_b, wb)
            w_wait(ib, buf_b, wb)

            @pl.when(ib + 2 < nb)
            def _():
                g_issue(ib + 2, buf_b, gb)

            return carry

        lax.fori_loop(0, nb // 2, body, 0)

    return gather


def kernel(ids, table):
    b, l = ids.shape
    gram = _gram(table)
    return _gather_fn(b, l)(gram, ids)


# speedup vs baseline: 1.8799x; 1.8422x over previous
import functools

import jax
import jax.numpy as jnp
from jax import lax
from jax.experimental import pallas as pl
from jax.experimental.pallas import tpu as pltpu
from jax.experimental.pallas import tpu_sc as plsc

_V = 1000
_D = 128
_CHUNK = 80


@functools.cache
def _gather_fn(n_tok):
    info = plsc.get_sparse_core_info()
    nc, ns = info.num_cores, info.num_subcores
    nw = nc * ns
    per_w = n_tok // nw
    n_steps = per_w // _CHUNK
    assert per_w * nw == n_tok and n_steps * _CHUNK == per_w and n_steps % 2 == 0
    mesh = plsc.VectorSubcoreMesh(core_axis_name="c", subcore_axis_name="s")

    @functools.partial(
        pl.kernel,
        mesh=mesh,
        out_type=jax.ShapeDtypeStruct((n_tok, _D), jnp.float32),
        scratch_types=[
            pltpu.VMEM((per_w,), jnp.int32),
            pltpu.VMEM((_CHUNK, _D), jnp.float32),
            pltpu.VMEM((_CHUNK, _D), jnp.float32),
            pltpu.SemaphoreType.DMA,
            pltpu.SemaphoreType.DMA,
            pltpu.SemaphoreType.DMA,
            pltpu.SemaphoreType.DMA,
        ],
    )
    def gather(table_hbm, ids_hbm, e_hbm, idx_v, buf_a, buf_b, ga, gb, wa, wb):
        wid = lax.axis_index("s") * nc + lax.axis_index("c")
        base = wid * per_w
        pltpu.sync_copy(ids_hbm.at[pl.ds(base, per_w)], idx_v)

        def g_issue(i, buf, sem):
            pltpu.async_copy(
                table_hbm.at[idx_v.at[pl.ds(i * _CHUNK, _CHUNK)]], buf, sem
            )

        def g_wait(i, buf, sem):
            pltpu.make_async_copy(
                table_hbm.at[idx_v.at[pl.ds(i * _CHUNK, _CHUNK)]], buf, sem
            ).wait()

        def w_issue(i, buf, sem):
            pltpu.async_copy(buf, e_hbm.at[pl.ds(base + i * _CHUNK, _CHUNK)], sem)

        def w_wait(i, buf, sem):
            pltpu.make_async_copy(
                buf, e_hbm.at[pl.ds(base + i * _CHUNK, _CHUNK)], sem
            ).wait()

        g_issue(0, buf_a, ga)
        g_issue(1, buf_b, gb)

        def body(j, carry):
            ia = 2 * j
            ib = ia + 1
            g_wait(ia, buf_a, ga)
            w_issue(ia, buf_a, wa)
            g_wait(ib, buf_b, gb)
            w_wait(ia, buf_a, wa)

            @pl.when(ia + 2 < n_steps)
            def _():
                g_issue(ia + 2, buf_a, ga)

            w_issue(ib, buf_b, wb)
            w_wait(ib, buf_b, wb)

            @pl.when(ib + 2 < n_steps)
            def _():
                g_issue(ib + 2, buf_b, gb)

            return carry

        lax.fori_loop(0, n_steps // 2, body, 0)

    return gather


_TB = 16


def _matmul_body(e_ref, t_ref, o_ref):
    e = e_ref[...].astype(jnp.bfloat16)
    t = t_ref[...].astype(jnp.bfloat16)
    o = lax.dot_general(
        e, t, (((1,), (1,)), ((), ())), preferred_element_type=jnp.float32
    )
    o_ref[...] = o.reshape(o_ref.shape)


def _matmul(e, table, b, l):
    n_tok = b * l
    tm = _TB * l
    return pl.pallas_call(
        _matmul_body,
        grid=(b // _TB,),
        in_specs=[
            pl.BlockSpec((tm, _D), lambda i: (i, 0)),
            pl.BlockSpec((_V, _D), lambda i: (0, 0)),
        ],
        out_specs=pl.BlockSpec((_TB, l, _V), lambda i: (i, 0, 0)),
        out_shape=jax.ShapeDtypeStruct((b, l, _V), jnp.float32),
    )(e, table)


def kernel(ids, table):
    b, l = ids.shape
    e = _gather_fn(b * l)(table, ids.reshape(-1))
    return _matmul(e, table, b, l)


# speedup vs baseline: 5.0692x; 2.6965x over previous
import functools

import jax
import jax.numpy as jnp
from jax import lax
from jax.experimental import pallas as pl
from jax.experimental.pallas import tpu as pltpu
from jax.experimental.pallas import tpu_sc as plsc

_V = 1000
_D = 128
_CHUNK = 80


@functools.cache
def _gather_fn(n_tok):
    info = plsc.get_sparse_core_info()
    nc, ns = info.num_cores, info.num_subcores
    nw = nc * ns
    per_w = n_tok // nw
    n_steps = per_w // _CHUNK
    assert per_w * nw == n_tok and n_steps * _CHUNK == per_w and n_steps % 2 == 0
    mesh = plsc.VectorSubcoreMesh(core_axis_name="c", subcore_axis_name="s")

    @functools.partial(
        pl.kernel,
        mesh=mesh,
        out_type=jax.ShapeDtypeStruct((n_tok, _D), jnp.float32),
        scratch_types=[
            pltpu.VMEM((per_w,), jnp.int32),
            pltpu.VMEM((_CHUNK, _D), jnp.float32),
            pltpu.VMEM((_CHUNK, _D), jnp.float32),
            pltpu.SemaphoreType.DMA,
            pltpu.SemaphoreType.DMA,
            pltpu.SemaphoreType.DMA,
            pltpu.SemaphoreType.DMA,
        ],
    )
    def gather(table_hbm, ids_hbm, e_hbm, idx_v, buf_a, buf_b, ga, gb, wa, wb):
        wid = lax.axis_index("s") * nc + lax.axis_index("c")
        base = wid * per_w
        pltpu.sync_copy(ids_hbm.at[pl.ds(base, per_w)], idx_v)

        def g_issue(i, buf, sem):
            pltpu.async_copy(
                table_hbm.at[idx_v.at[pl.ds(i * _CHUNK, _CHUNK)]], buf, sem
            )

        def g_wait(i, buf, sem):
            pltpu.make_async_copy(
                table_hbm.at[idx_v.at[pl.ds(i * _CHUNK, _CHUNK)]], buf, sem
            ).wait()

        def w_issue(i, buf, sem):
            pltpu.async_copy(buf, e_hbm.at[pl.ds(base + i * _CHUNK, _CHUNK)], sem)

        def w_wait(i, buf, sem):
            pltpu.make_async_copy(
                buf, e_hbm.at[pl.ds(base + i * _CHUNK, _CHUNK)], sem
            ).wait()

        g_issue(0, buf_a, ga)
        g_issue(1, buf_b, gb)

        def body(j, carry):
            ia = 2 * j
            ib = ia + 1
            g_wait(ia, buf_a, ga)
            w_issue(ia, buf_a, wa)
            g_wait(ib, buf_b, gb)
            w_wait(ia, buf_a, wa)

            @pl.when(ia + 2 < n_steps)
            def _():
                g_issue(ia + 2, buf_a, ga)

            w_issue(ib, buf_b, wb)
            w_wait(ib, buf_b, wb)

            @pl.when(ib + 2 < n_steps)
            def _():
                g_issue(ib + 2, buf_b, gb)

            return carry

        lax.fori_loop(0, n_steps // 2, body, 0)

    return gather


def _matmul_body(e_ref, t_ref, o_ref):
    e = e_ref[0].astype(jnp.bfloat16)
    t = t_ref[...].astype(jnp.bfloat16)
    o = lax.dot_general(
        t, e, (((1,), (1,)), ((), ())), preferred_element_type=jnp.float32
    )
    o_ref[0] = o


def _matmul(e2, table, b, l):
    return pl.pallas_call(
        _matmul_body,
        grid=(l,),
        in_specs=[
            pl.BlockSpec((1, b, _D), lambda i: (i, 0, 0)),
            pl.BlockSpec((_V, _D), lambda i: (0, 0)),
        ],
        out_specs=pl.BlockSpec((1, _V, b), lambda i: (i, 0, 0)),
        out_shape=jax.ShapeDtypeStruct((l, _V, b), jnp.float32),
    )(e2, table)


def kernel(ids, table):
    b, l = ids.shape
    ids_t = jnp.transpose(ids)
    e = _gather_fn(b * l)(table, ids_t.reshape(-1))
    e2 = e.reshape(l, b, _D)
    p = _matmul(e2, table, b, l)
    return jnp.transpose(p, (2, 0, 1))


# speedup vs baseline: 5.0994x; 1.0060x over previous
import functools

import jax
import jax.numpy as jnp
from jax import lax
from jax.experimental import pallas as pl
from jax.experimental.pallas import tpu as pltpu
from jax.experimental.pallas import tpu_sc as plsc

_V = 1000
_D = 128
_CHUNK = 80


@functools.cache
def _gather_fn(n_tok):
    info = plsc.get_sparse_core_info()
    nc, ns = info.num_cores, info.num_subcores
    nw = nc * ns
    per_w = n_tok // nw
    n_steps = per_w // _CHUNK
    assert per_w * nw == n_tok and n_steps * _CHUNK == per_w and n_steps % 2 == 0
    mesh = plsc.VectorSubcoreMesh(core_axis_name="c", subcore_axis_name="s")

    @functools.partial(
        pl.kernel,
        mesh=mesh,
        out_type=jax.ShapeDtypeStruct((n_tok, _D), jnp.float32),
        scratch_types=[
            pltpu.VMEM((per_w,), jnp.int32),
            pltpu.VMEM((_CHUNK, _D), jnp.float32),
            pltpu.VMEM((_CHUNK, _D), jnp.float32),
            pltpu.SemaphoreType.DMA,
            pltpu.SemaphoreType.DMA,
            pltpu.SemaphoreType.DMA,
            pltpu.SemaphoreType.DMA,
        ],
    )
    def gather(table_hbm, ids_hbm, e_hbm, idx_v, buf_a, buf_b, ga, gb, wa, wb):
        wid = lax.axis_index("s") * nc + lax.axis_index("c")
        base = wid * per_w
        pltpu.sync_copy(ids_hbm.at[pl.ds(base, per_w)], idx_v)

        def g_issue(i, buf, sem):
            pltpu.async_copy(
                table_hbm.at[idx_v.at[pl.ds(i * _CHUNK, _CHUNK)]], buf, sem
            )

        def g_wait(i, buf, sem):
            pltpu.make_async_copy(
                table_hbm.at[idx_v.at[pl.ds(i * _CHUNK, _CHUNK)]], buf, sem
            ).wait()

        def w_issue(i, buf, sem):
            pltpu.async_copy(buf, e_hbm.at[pl.ds(base + i * _CHUNK, _CHUNK)], sem)

        def w_wait(i, buf, sem):
            pltpu.make_async_copy(
                buf, e_hbm.at[pl.ds(base + i * _CHUNK, _CHUNK)], sem
            ).wait()

        g_issue(0, buf_a, ga)
        g_issue(1, buf_b, gb)

        def body(j, carry):
            ia = 2 * j
            ib = ia + 1
            g_wait(ia, buf_a, ga)
            w_issue(ia, buf_a, wa)
            g_wait(ib, buf_b, gb)
            w_wait(ia, buf_a, wa)

            @pl.when(ia + 2 < n_steps)
            def _():
                g_issue(ia + 2, buf_a, ga)

            w_issue(ib, buf_b, wb)
            w_wait(ib, buf_b, wb)

            @pl.when(ib + 2 < n_steps)
            def _():
                g_issue(ib + 2, buf_b, gb)

            return carry

        lax.fori_loop(0, n_steps // 2, body, 0)

    return gather


def _matmul_body(e_ref, t_ref, o_ref):
    e = e_ref[0].astype(jnp.bfloat16)
    t = t_ref[...].astype(jnp.bfloat16)
    o = lax.dot_general(
        t, e, (((1,), (1,)), ((), ())), preferred_element_type=jnp.float32
    )
    o_ref[0] = o


def _matmul_alias_body(e_ref, t_ref, _p_ref, o_ref):
    _matmul_body(e_ref, t_ref, o_ref)


def _matmul_first(e2, table, b, l, lh):
    return pl.pallas_call(
        _matmul_body,
        grid=(lh,),
        in_specs=[
            pl.BlockSpec((1, b, _D), lambda i: (i, 0, 0)),
            pl.BlockSpec((_V, _D), lambda i: (0, 0)),
        ],
        out_specs=pl.BlockSpec((1, _V, b), lambda i: (i, 0, 0)),
        out_shape=jax.ShapeDtypeStruct((l, _V, b), jnp.float32),
    )(e2, table)


def _matmul_second(e2, table, p, b, l, lh):
    return pl.pallas_call(
        _matmul_alias_body,
        grid=(l - lh,),
        in_specs=[
            pl.BlockSpec((1, b, _D), lambda i: (i, 0, 0)),
            pl.BlockSpec((_V, _D), lambda i: (0, 0)),
            pl.BlockSpec(memory_space=pl.ANY),
        ],
        out_specs=pl.BlockSpec((1, _V, b), lambda i: (i + lh, 0, 0)),
        out_shape=jax.ShapeDtypeStruct((l, _V, b), jnp.float32),
        input_output_aliases={2: 0},
    )(e2, table, p)


def kernel(ids, table):
    b, l = ids.shape
    lh = l // 2
    ids_t = jnp.transpose(ids)
    idx_a = ids_t[:lh].reshape(-1)
    idx_b = ids_t[lh:].reshape(-1)
    e_a = _gather_fn(lh * b)(table, idx_a).reshape(lh, b, _D)
    e_b = _gather_fn((l - lh) * b)(table, idx_b).reshape(l - lh, b, _D)
    p = _matmul_first(e_a, table, b, l, lh)
    p = _matmul_second(e_b, table, p, b, l, lh)
    return jnp.transpose(p, (2, 0, 1))


# speedup vs baseline: 5.1577x; 1.0114x over previous
import functools

import jax
import jax.numpy as jnp
from jax import lax
from jax.experimental import pallas as pl
from jax.experimental.pallas import tpu as pltpu
from jax.experimental.pallas import tpu_sc as plsc

_V = 1000
_D = 128
_CHUNK = 80


@functools.cache
def _gather_fn(n_tok):
    info = plsc.get_sparse_core_info()
    nc, ns = info.num_cores, info.num_subcores
    nw = nc * ns
    per_w = n_tok // nw
    n_steps = per_w // _CHUNK
    assert per_w * nw == n_tok and n_steps * _CHUNK == per_w and n_steps % 2 == 0
    mesh = plsc.VectorSubcoreMesh(core_axis_name="c", subcore_axis_name="s")

    @functools.partial(
        pl.kernel,
        mesh=mesh,
        out_type=jax.ShapeDtypeStruct((n_tok, _D), jnp.float32),
        scratch_types=[
            pltpu.VMEM((per_w,), jnp.int32),
            pltpu.VMEM((_CHUNK, _D), jnp.float32),
            pltpu.VMEM((_CHUNK, _D), jnp.float32),
            pltpu.SemaphoreType.DMA,
            pltpu.SemaphoreType.DMA,
            pltpu.SemaphoreType.DMA,
            pltpu.SemaphoreType.DMA,
        ],
    )
    def gather(table_hbm, ids_hbm, e_hbm, idx_v, buf_a, buf_b, ga, gb, wa, wb):
        wid = lax.axis_index("s") * nc + lax.axis_index("c")
        base = wid * per_w
        pltpu.sync_copy(ids_hbm.at[pl.ds(base, per_w)], idx_v)

        def g_issue(i, buf, sem):
            pltpu.async_copy(
                table_hbm.at[idx_v.at[pl.ds(i * _CHUNK, _CHUNK)]], buf, sem
            )

        def g_wait(i, buf, sem):
            pltpu.make_async_copy(
                table_hbm.at[idx_v.at[pl.ds(i * _CHUNK, _CHUNK)]], buf, sem
            ).wait()

        def w_issue(i, buf, sem):
            pltpu.async_copy(buf, e_hbm.at[pl.ds(base + i * _CHUNK, _CHUNK)], sem)

        def w_wait(i, buf, sem):
            pltpu.make_async_copy(
                buf, e_hbm.at[pl.ds(base + i * _CHUNK, _CHUNK)], sem
            ).wait()

        g_issue(0, buf_a, ga)
        g_issue(1, buf_b, gb)

        def body(j, carry):
            ia = 2 * j
            ib = ia + 1
            g_wait(ia, buf_a, ga)
            w_issue(ia, buf_a, wa)
            g_wait(ib, buf_b, gb)
            w_wait(ia, buf_a, wa)

            @pl.when(ia + 2 < n_steps)
            def _():
                g_issue(ia + 2, buf_a, ga)

            w_issue(ib, buf_b, wb)
            w_wait(ib, buf_b, wb)

            @pl.when(ib + 2 < n_steps)
            def _():
                g_issue(ib + 2, buf_b, gb)

            return carry

        lax.fori_loop(0, n_steps // 2, body, 0)

    return gather


def _matmul_body(e_ref, t_ref, o_ref):
    e = e_ref[0].astype(jnp.bfloat16)
    t = t_ref[...].astype(jnp.bfloat16)
    o = lax.dot_general(
        t, e, (((1,), (1,)), ((), ())), preferred_element_type=jnp.float32
    )
    o_ref[0] = o


def _matmul_alias_body(e_ref, t_ref, _p_ref, o_ref):
    _matmul_body(e_ref, t_ref, o_ref)


def _matmul_chunk(e2, table, p, b, l, off, lc):
    if p is None:
        return pl.pallas_call(
            _matmul_body,
            grid=(lc,),
            in_specs=[
                pl.BlockSpec((1, b, _D), lambda i: (i, 0, 0)),
                pl.BlockSpec((_V, _D), lambda i: (0, 0)),
            ],
            out_specs=pl.BlockSpec((1, _V, b), lambda i: (i + off, 0, 0)),
            out_shape=jax.ShapeDtypeStruct((l, _V, b), jnp.float32),
        )(e2, table)
    return pl.pallas_call(
        _matmul_alias_body,
        grid=(lc,),
        in_specs=[
            pl.BlockSpec((1, b, _D), lambda i: (i, 0, 0)),
            pl.BlockSpec((_V, _D), lambda i: (0, 0)),
            pl.BlockSpec(memory_space=pl.ANY),
        ],
        out_specs=pl.BlockSpec((1, _V, b), lambda i: (i + off, 0, 0)),
        out_shape=jax.ShapeDtypeStruct((l, _V, b), jnp.float32),
        input_output_aliases={2: 0},
    )(e2, table, p)


_SPLITS = (10, 15, 25)


def kernel(ids, table):
    b, l = ids.shape
    assert sum(_SPLITS) == l
    ids_t = jnp.transpose(ids)
    chunks = []
    off = 0
    for lc in _SPLITS:
        idx = ids_t[off:off + lc].reshape(-1)
        e = _gather_fn(lc * b)(table, idx).reshape(lc, b, _D)
        chunks.append((off, lc, e))
        off += lc
    p = None
    for off, lc, e in chunks:
        p = _matmul_chunk(e, table, p, b, l, off, lc)
    return jnp.transpose(p, (2, 0, 1))
